# baseline (speedup 1.0000x reference)
import functools

import jax
import jax.numpy as jnp
from jax import lax
from jax.experimental import pallas as pl
from jax.experimental.pallas import tpu as pltpu
from jax.experimental.pallas import tpu_sc as plsc

_NUM_CORES = 2
_NUM_SUBCORES = 16
_NUM_WORKERS = _NUM_CORES * _NUM_SUBCORES

_LANES = 16
_NBUF = 2


@jax.jit
def _embedding_lookup(table, ids_t):
    hist, batch = ids_t.shape
    depth = table.shape[1]
    n_dt = depth // 8
    n_cb = batch // 128
    cb_per_w = n_cb // _NUM_WORKERS
    bw = 128 * cb_per_w
    n_groups = hist // _NBUF

    mesh = plsc.VectorSubcoreMesh(
        core_axis_name="c",
        subcore_axis_name="s",
        num_cores=_NUM_CORES,
        num_subcores=_NUM_SUBCORES,
    )

    @functools.partial(
        pl.kernel,
        mesh=mesh,
        out_type=jax.ShapeDtypeStruct((hist, n_dt, n_cb, 8, 128),
                                      table.dtype),
        scratch_types=(
            [pltpu.VMEM((bw,), jnp.int32) for _ in range(_NBUF)]
            + [pltpu.VMEM((bw, depth), table.dtype) for _ in range(_NBUF)]
            + [pltpu.VMEM((cb_per_w, n_dt, 8, 128), table.dtype)
               for _ in range(_NBUF)]
            + [pltpu.SemaphoreType.DMA((_NBUF,)),
               pltpu.SemaphoreType.DMA((_NBUF,))]
        ),
        compiler_params=pltpu.CompilerParams(use_tc_tiling_on_sc=False,
                                             needs_layout_passes=False),
    )
    def emb_kernel(table_hbm, idx_hbm, out_hbm, *scratch):
        idx_v = scratch[:_NBUF]
        rows_v = scratch[_NBUF:2 * _NBUF]
        panel_v = scratch[2 * _NBUF:3 * _NBUF]
        gsem, osem = scratch[3 * _NBUF], scratch[3 * _NBUF + 1]
        wid = lax.axis_index("s") * _NUM_CORES + lax.axis_index("c")
        woff = wid * bw
        cbase = wid * cb_per_w
        iota16 = lax.iota(jnp.int32, _LANES)

        def start(t, s):
            pltpu.sync_copy(idx_hbm.at[t, pl.ds(woff, bw)], idx_v[s])
            pltpu.async_copy(table_hbm.at[idx_v[s]], rows_v[s], gsem.at[s])

        def wait_gather(s):
            pltpu.make_async_copy(table_hbm.at[idx_v[s]], rows_v[s],
                                  gsem.at[s]).wait()

        def transpose(s):
            for cc in range(cb_per_w):
                for r in range(n_dt):
                    def dr_body(dr, carry, cc=cc, r=r):
                        col = jnp.full((_LANES,), 8 * r + dr, jnp.int32)
                        for dc0 in range(0, 128, _LANES):
                            rowv = iota16 + (128 * cc + dc0)
                            vec = plsc.load_gather(rows_v[s], [rowv, col])
                            panel_v[s][cc, r, dr, pl.ds(dc0, _LANES)] = vec
                        return carry
                    lax.fori_loop(0, 8, dr_body, 0)

        def fire_out(t, s):
            for cc in range(cb_per_w):
                pltpu.async_copy(panel_v[s].at[cc],
                                 out_hbm.at[t, :, cbase + cc],
                                 osem.at[s])

        def wait_out(t, s):
            for cc in range(cb_per_w):
                pltpu.make_async_copy(panel_v[s].at[cc],
                                      out_hbm.at[t, :, cbase + cc],
                                      osem.at[s]).wait()

        for s in range(_NBUF):
            start(s, s)

        def body(g, carry):
            for s in range(_NBUF):
                t_proc = (g - 1) * _NBUF + s
                t_next = g * _NBUF + s
                wait_gather(s)

                @pl.when(g >= 2)
                def _():
                    wait_out(t_proc - _NBUF, s)

                transpose(s)
                fire_out(t_proc, s)
                start(t_next, s)
            return carry

        lax.fori_loop(1, n_groups, body, 0)

        for s in range(_NBUF):
            t_proc = (n_groups - 1) * _NBUF + s
            wait_gather(s)
            wait_out(t_proc - _NBUF, s)
            transpose(s)
            fire_out(t_proc, s)
        for s in range(_NBUF):
            t_proc = (n_groups - 1) * _NBUF + s
            wait_out(t_proc, s)

    return emb_kernel(table, ids_t)


def kernel(words_as_ids, embedding_table):
    batch, hist = words_as_ids.shape
    depth = embedding_table.shape[1]
    ids_t = words_as_ids.astype(jnp.int32).T
    out5 = _embedding_lookup(embedding_table, ids_t)
    out = out5.transpose(0, 1, 3, 2, 4).reshape(hist, depth, batch)
    return out.transpose(2, 0, 1)


# speedup vs baseline: 2.4861x; 2.4861x over previous
import functools

import jax
import jax.numpy as jnp
from jax import lax
from jax.experimental import pallas as pl
from jax.experimental.pallas import tpu as pltpu
from jax.experimental.pallas import tpu_sc as plsc

_NUM_CORES = 2
_NUM_SUBCORES = 16
_NUM_WORKERS = _NUM_CORES * _NUM_SUBCORES

_LANES = 16
_NBUF = 2


@jax.jit
def _embedding_lookup(table, ids_t):
    hist, batch = ids_t.shape
    depth = table.shape[1]
    n_dt = depth // 8
    n_cb = batch // 128
    cb_per_w = n_cb // _NUM_WORKERS
    bw = 128 * cb_per_w
    n_groups = hist // _NBUF

    mesh = plsc.VectorSubcoreMesh(
        core_axis_name="c",
        subcore_axis_name="s",
        num_cores=_NUM_CORES,
        num_subcores=_NUM_SUBCORES,
    )

    @functools.partial(
        pl.kernel,
        mesh=mesh,
        out_type=jax.ShapeDtypeStruct((hist, n_dt, n_cb, 8, 128),
                                      table.dtype),
        scratch_types=(
            [pltpu.VMEM((bw,), jnp.int32) for _ in range(_NBUF)]
            + [pltpu.VMEM((bw, depth), table.dtype) for _ in range(_NBUF)]
            + [pltpu.VMEM((cb_per_w, depth, 129), table.dtype)
               for _ in range(_NBUF)]
            + [pltpu.SemaphoreType.DMA((_NBUF,)),
               pltpu.SemaphoreType.DMA((_NBUF,))]
        ),
        compiler_params=pltpu.CompilerParams(use_tc_tiling_on_sc=False,
                                             needs_layout_passes=False),
    )
    def emb_kernel(table_hbm, idx_hbm, out_hbm, *scratch):
        idx_v = scratch[:_NBUF]
        rows_v = scratch[_NBUF:2 * _NBUF]
        panel_v = scratch[2 * _NBUF:3 * _NBUF]
        gsem, osem = scratch[3 * _NBUF], scratch[3 * _NBUF + 1]
        wid = lax.axis_index("s") * _NUM_CORES + lax.axis_index("c")
        woff = wid * bw
        cbase = wid * cb_per_w
        iota16 = lax.iota(jnp.int32, _LANES)

        def start(t, s):
            pltpu.sync_copy(idx_hbm.at[t, pl.ds(woff, bw)], idx_v[s])
            pltpu.async_copy(table_hbm.at[idx_v[s]], rows_v[s], gsem.at[s])

        def wait_gather(s):
            pltpu.make_async_copy(table_hbm.at[idx_v[s]], rows_v[s],
                                  gsem.at[s]).wait()

        def transpose(s):
            for cc in range(cb_per_w):
                ccv = jnp.full((_LANES,), cc, jnp.int32)

                def b_body(b2, carry, cc=cc, ccv=ccv):
                    row = 128 * cc + b2
                    dcv = jnp.full((_LANES,), b2, jnp.int32)
                    v0 = rows_v[s][row, pl.ds(0, _LANES)]
                    v1 = rows_v[s][row, pl.ds(_LANES, _LANES)]
                    plsc.store_scatter(panel_v[s], [ccv, iota16, dcv], v0)
                    plsc.store_scatter(panel_v[s], [ccv, iota16 + _LANES,
                                                    dcv], v1)
                    return carry

                lax.fori_loop(0, 128, b_body, 0)

        def fire_out(t, s):
            for cc in range(cb_per_w):
                for r in range(n_dt):
                    pltpu.async_copy(
                        panel_v[s].at[cc, pl.ds(8 * r, 8), pl.ds(0, 128)],
                        out_hbm.at[t, r, cbase + cc], osem.at[s])

        def wait_out(t, s):
            for cc in range(cb_per_w):
                for r in range(n_dt):
                    pltpu.make_async_copy(
                        panel_v[s].at[cc, pl.ds(8 * r, 8), pl.ds(0, 128)],
                        out_hbm.at[t, r, cbase + cc], osem.at[s]).wait()

        for s in range(_NBUF):
            start(s, s)

        def body(g, carry):
            for s in range(_NBUF):
                t_proc = (g - 1) * _NBUF + s
                t_next = g * _NBUF + s
                wait_gather(s)

                @pl.when(g >= 2)
                def _():
                    wait_out(t_proc - _NBUF, s)

                transpose(s)
                fire_out(t_proc, s)
                start(t_next, s)
            return carry

        lax.fori_loop(1, n_groups, body, 0)

        for s in range(_NBUF):
            t_proc = (n_groups - 1) * _NBUF + s
            wait_gather(s)
            wait_out(t_proc - _NBUF, s)
            transpose(s)
            fire_out(t_proc, s)
        for s in range(_NBUF):
            t_proc = (n_groups - 1) * _NBUF + s
            wait_out(t_proc, s)

    return emb_kernel(table, ids_t)


def kernel(words_as_ids, embedding_table):
    batch, hist = words_as_ids.shape
    depth = embedding_table.shape[1]
    ids_t = words_as_ids.astype(jnp.int32).T
    out5 = _embedding_lookup(embedding_table, ids_t)
    out = out5.transpose(0, 1, 3, 2, 4).reshape(hist, depth, batch)
    return out.transpose(2, 0, 1)


# speedup vs baseline: 2.5681x; 1.0330x over previous
import functools

import jax
import jax.numpy as jnp
from jax import lax
from jax.experimental import pallas as pl
from jax.experimental.pallas import tpu as pltpu
from jax.experimental.pallas import tpu_sc as plsc

_NUM_CORES = 2
_NUM_SUBCORES = 16
_NUM_WORKERS = _NUM_CORES * _NUM_SUBCORES

_LANES = 16
_NBUF = 2


@jax.jit
def _embedding_lookup(table, ids_t):
    hist, batch = ids_t.shape
    depth = table.shape[1]
    n_dt = depth // 8
    n_cb = batch // 128
    cb_per_w = n_cb // _NUM_WORKERS
    bw = 128 * cb_per_w
    n_groups = hist // _NBUF

    mesh = plsc.VectorSubcoreMesh(
        core_axis_name="c",
        subcore_axis_name="s",
        num_cores=_NUM_CORES,
        num_subcores=_NUM_SUBCORES,
    )

    @functools.partial(
        pl.kernel,
        mesh=mesh,
        out_type=jax.ShapeDtypeStruct((hist, n_dt, n_cb, 8, 128),
                                      table.dtype),
        scratch_types=(
            [pltpu.VMEM((bw,), jnp.int32) for _ in range(_NBUF)]
            + [pltpu.VMEM((bw, depth), table.dtype) for _ in range(_NBUF)]
            + [pltpu.VMEM((cb_per_w, depth, 129), table.dtype)
               for _ in range(_NBUF)]
            + [pltpu.SemaphoreType.DMA((_NBUF,)),
               pltpu.SemaphoreType.DMA((_NBUF,))]
        ),
        compiler_params=pltpu.CompilerParams(use_tc_tiling_on_sc=False,
                                             needs_layout_passes=False),
    )
    def emb_kernel(table_hbm, idx_hbm, out_hbm, *scratch):
        idx_v = scratch[:_NBUF]
        rows_v = scratch[_NBUF:2 * _NBUF]
        panel_v = scratch[2 * _NBUF:3 * _NBUF]
        gsem, osem = scratch[3 * _NBUF], scratch[3 * _NBUF + 1]
        wid = lax.axis_index("s") * _NUM_CORES + lax.axis_index("c")
        woff = wid * bw
        cbase = wid * cb_per_w
        iota_lo = lax.iota(jnp.int32, _LANES)
        iota_hi = iota_lo + _LANES

        def start(t, s):
            pltpu.sync_copy(idx_hbm.at[t, pl.ds(woff, bw)], idx_v[s])
            pltpu.async_copy(table_hbm.at[idx_v[s]], rows_v[s], gsem.at[s])

        def wait_gather(s):
            pltpu.make_async_copy(table_hbm.at[idx_v[s]], rows_v[s],
                                  gsem.at[s]).wait()

        def transpose(s):
            for cc in range(cb_per_w):
                ccv = jnp.full((_LANES,), cc, jnp.int32)

                def b_body(b8, carry, cc=cc, ccv=ccv):
                    for k in range(8):
                        b2 = b8 * 8 + k
                        row = 128 * cc + b2
                        dcv = jnp.full((_LANES,), b2, jnp.int32)
                        v0 = rows_v[s][row, pl.ds(0, _LANES)]
                        v1 = rows_v[s][row, pl.ds(_LANES, _LANES)]
                        plsc.store_scatter(panel_v[s], [ccv, iota_lo, dcv],
                                           v0)
                        plsc.store_scatter(panel_v[s], [ccv, iota_hi, dcv],
                                           v1)
                    return carry

                lax.fori_loop(0, 16, b_body, 0)

        def fire_out(t, s):
            for cc in range(cb_per_w):
                for r in range(n_dt):
                    pltpu.async_copy(
                        panel_v[s].at[cc, pl.ds(8 * r, 8), pl.ds(0, 128)],
                        out_hbm.at[t, r, cbase + cc], osem.at[s])

        def wait_out(t, s):
            for cc in range(cb_per_w):
                for r in range(n_dt):
                    pltpu.make_async_copy(
                        panel_v[s].at[cc, pl.ds(8 * r, 8), pl.ds(0, 128)],
                        out_hbm.at[t, r, cbase + cc], osem.at[s]).wait()

        for s in range(_NBUF):
            start(s, s)

        def body(g, carry):
            for s in range(_NBUF):
                t_proc = (g - 1) * _NBUF + s
                t_next = g * _NBUF + s
                wait_gather(s)

                @pl.when(g >= 2)
                def _():
                    wait_out(t_proc - _NBUF, s)

                transpose(s)
                fire_out(t_proc, s)
                start(t_next, s)
            return carry

        lax.fori_loop(1, n_groups, body, 0)

        for s in range(_NBUF):
            t_proc = (n_groups - 1) * _NBUF + s
            wait_gather(s)
            wait_out(t_proc - _NBUF, s)
            transpose(s)
            fire_out(t_proc, s)
        for s in range(_NBUF):
            t_proc = (n_groups - 1) * _NBUF + s
            wait_out(t_proc, s)

    return emb_kernel(table, ids_t)


def kernel(words_as_ids, embedding_table):
    batch, hist = words_as_ids.shape
    depth = embedding_table.shape[1]
    ids_t = words_as_ids.astype(jnp.int32).T
    out5 = _embedding_lookup(embedding_table, ids_t)
    out = out5.transpose(0, 1, 3, 2, 4).reshape(hist, depth, batch)
    return out.transpose(2, 0, 1)


# speedup vs baseline: 2.6162x; 1.0187x over previous
import functools

import jax
import jax.numpy as jnp
from jax import lax
from jax.experimental import pallas as pl
from jax.experimental.pallas import tpu as pltpu
from jax.experimental.pallas import tpu_sc as plsc

_NUM_CORES = 2
_NUM_SUBCORES = 16
_NUM_WORKERS = _NUM_CORES * _NUM_SUBCORES

_LANES = 16
_NBUF = 2


def _table_rows(table_t, block_v):
    depth, num_v = table_t.shape
    pack = 128 // depth
    grid = (num_v + block_v - 1) // block_v

    def body(x_ref, o_ref):
        t4 = x_ref[...].T.reshape(block_v // pack, pack, depth)
        for l in range(pack):
            o_ref[:, depth * l:depth * (l + 1)] = t4[:, l, :]

    return pl.pallas_call(
        body,
        grid=(grid,),
        in_specs=[pl.BlockSpec((depth, block_v), lambda j: (0, j))],
        out_specs=pl.BlockSpec((block_v // pack, 128), lambda j: (j, 0)),
        out_shape=jax.ShapeDtypeStruct((num_v * depth // 128, 128),
                                       table_t.dtype),
    )(table_t)


@jax.jit
def _embedding_lookup(table, ids_t):
    hist, batch = ids_t.shape
    depth = table.shape[1]
    n_dt = depth // 8
    n_cb = batch // 128
    cb_per_w = n_cb // _NUM_WORKERS
    bw = 128 * cb_per_w
    n_groups = hist // _NBUF

    mesh = plsc.VectorSubcoreMesh(
        core_axis_name="c",
        subcore_axis_name="s",
        num_cores=_NUM_CORES,
        num_subcores=_NUM_SUBCORES,
    )

    @functools.partial(
        pl.kernel,
        mesh=mesh,
        out_type=jax.ShapeDtypeStruct((hist, n_dt, n_cb, 8, 128),
                                      table.dtype),
        scratch_types=(
            [pltpu.VMEM((bw,), jnp.int32) for _ in range(_NBUF)]
            + [pltpu.VMEM((bw, depth), table.dtype) for _ in range(_NBUF)]
            + [pltpu.VMEM((cb_per_w, depth, 129), table.dtype)
               for _ in range(_NBUF)]
            + [pltpu.SemaphoreType.DMA((_NBUF,)),
               pltpu.SemaphoreType.DMA((_NBUF,))]
        ),
        compiler_params=pltpu.CompilerParams(use_tc_tiling_on_sc=False,
                                             needs_layout_passes=False),
    )
    def emb_kernel(table_hbm, idx_hbm, out_hbm, *scratch):
        idx_v = scratch[:_NBUF]
        rows_v = scratch[_NBUF:2 * _NBUF]
        panel_v = scratch[2 * _NBUF:3 * _NBUF]
        gsem, osem = scratch[3 * _NBUF], scratch[3 * _NBUF + 1]
        wid = lax.axis_index("s") * _NUM_CORES + lax.axis_index("c")
        woff = wid * bw
        cbase = wid * cb_per_w
        iota_lo = lax.iota(jnp.int32, _LANES)
        iota_hi = iota_lo + _LANES

        def start(t, s):
            pltpu.sync_copy(idx_hbm.at[t, pl.ds(woff, bw)], idx_v[s])
            pltpu.async_copy(table_hbm.at[idx_v[s]], rows_v[s], gsem.at[s])

        def wait_gather(s):
            pltpu.make_async_copy(table_hbm.at[idx_v[s]], rows_v[s],
                                  gsem.at[s]).wait()

        def transpose(s):
            for cc in range(cb_per_w):
                ccv = jnp.full((_LANES,), cc, jnp.int32)

                def b_body(b8, carry, cc=cc, ccv=ccv):
                    for k in range(8):
                        b2 = b8 * 8 + k
                        row = 128 * cc + b2
                        dcv = jnp.full((_LANES,), b2, jnp.int32)
                        v0 = rows_v[s][row, pl.ds(0, _LANES)]
                        v1 = rows_v[s][row, pl.ds(_LANES, _LANES)]
                        plsc.store_scatter(panel_v[s], [ccv, iota_lo, dcv],
                                           v0)
                        plsc.store_scatter(panel_v[s], [ccv, iota_hi, dcv],
                                           v1)
                    return carry

                lax.fori_loop(0, 16, b_body, 0)

        def fire_out(t, s):
            for cc in range(cb_per_w):
                for r in range(n_dt):
                    pltpu.async_copy(
                        panel_v[s].at[cc, pl.ds(8 * r, 8), pl.ds(0, 128)],
                        out_hbm.at[t, r, cbase + cc], osem.at[s])

        def wait_out(t, s):
            for cc in range(cb_per_w):
                for r in range(n_dt):
                    pltpu.make_async_copy(
                        panel_v[s].at[cc, pl.ds(8 * r, 8), pl.ds(0, 128)],
                        out_hbm.at[t, r, cbase + cc], osem.at[s]).wait()

        for s in range(_NBUF):
            start(s, s)

        def body(g, carry):
            for s in range(_NBUF):
                t_proc = (g - 1) * _NBUF + s
                t_next = g * _NBUF + s
                wait_gather(s)

                @pl.when(g >= 2)
                def _():
                    wait_out(t_proc - _NBUF, s)

                transpose(s)
                fire_out(t_proc, s)
                start(t_next, s)
            return carry

        lax.fori_loop(1, n_groups, body, 0)

        for s in range(_NBUF):
            t_proc = (n_groups - 1) * _NBUF + s
            wait_gather(s)
            wait_out(t_proc - _NBUF, s)
            transpose(s)
            fire_out(t_proc, s)
        for s in range(_NBUF):
            t_proc = (n_groups - 1) * _NBUF + s
            wait_out(t_proc, s)

    return emb_kernel(table, ids_t)


def kernel(words_as_ids, embedding_table):
    batch, hist = words_as_ids.shape
    depth = embedding_table.shape[1]
    ids_t = words_as_ids.astype(jnp.int32).T
    num_v = embedding_table.shape[0]
    table_rm = _table_rows(embedding_table.T, 2048).reshape(num_v, depth)
    out5 = _embedding_lookup(table_rm, ids_t)
    out = out5.transpose(0, 1, 3, 2, 4).reshape(hist, depth, batch)
    return out.transpose(2, 0, 1)


# speedup vs baseline: 2.9053x; 1.1105x over previous
import functools

import jax
import jax.numpy as jnp
from jax import lax
from jax.experimental import pallas as pl
from jax.experimental.pallas import tpu as pltpu
from jax.experimental.pallas import tpu_sc as plsc

_NUM_CORES = 2
_NUM_SUBCORES = 16
_NUM_WORKERS = _NUM_CORES * _NUM_SUBCORES

_LANES = 16
_NBUF = 2


def _table_rows(table_t, block_v):
    depth, num_v = table_t.shape
    pack = 128 // depth
    grid = (num_v + block_v - 1) // block_v

    def body(x_ref, o_ref):
        t4 = x_ref[...].T.reshape(block_v // pack, pack, depth)
        for l in range(pack):
            o_ref[:, depth * l:depth * (l + 1)] = t4[:, l, :]

    return pl.pallas_call(
        body,
        grid=(grid,),
        in_specs=[pl.BlockSpec((depth, block_v), lambda j: (0, j))],
        out_specs=pl.BlockSpec((block_v // pack, 128), lambda j: (j, 0)),
        out_shape=jax.ShapeDtypeStruct((num_v * depth // 128, 128),
                                       table_t.dtype),
    )(table_t)


@jax.jit
def _embedding_lookup(table, ids_t):
    hist, batch = ids_t.shape
    depth = table.shape[1]
    n_dt = depth // 8
    n_cb = batch // 128
    cb_per_w = n_cb // _NUM_WORKERS
    bw = 128 * cb_per_w
    n_groups = hist // _NBUF

    mesh = plsc.VectorSubcoreMesh(
        core_axis_name="c",
        subcore_axis_name="s",
        num_cores=_NUM_CORES,
        num_subcores=_NUM_SUBCORES,
    )

    @functools.partial(
        pl.kernel,
        mesh=mesh,
        out_type=jax.ShapeDtypeStruct((hist, n_dt, n_cb, 8, 128),
                                      table.dtype),
        scratch_types=(
            [pltpu.VMEM((bw,), jnp.int32) for _ in range(_NBUF)]
            + [pltpu.VMEM((bw, depth), table.dtype) for _ in range(_NBUF)]
            + [pltpu.VMEM((cb_per_w, depth, 129), table.dtype)
               for _ in range(_NBUF)]
            + [pltpu.SemaphoreType.DMA((_NBUF,)),
               pltpu.SemaphoreType.DMA((_NBUF,))]
        ),
        compiler_params=pltpu.CompilerParams(use_tc_tiling_on_sc=False,
                                             needs_layout_passes=False),
    )
    def emb_kernel(table_hbm, idx_hbm, out_hbm, *scratch):
        idx_v = scratch[:_NBUF]
        rows_v = scratch[_NBUF:2 * _NBUF]
        panel_v = scratch[2 * _NBUF:3 * _NBUF]
        gsem, osem = scratch[3 * _NBUF], scratch[3 * _NBUF + 1]
        wid = lax.axis_index("s") * _NUM_CORES + lax.axis_index("c")
        woff = wid * bw
        cbase = wid * cb_per_w
        iota_lo = lax.iota(jnp.int32, _LANES)
        iota_hi = iota_lo + _LANES

        def start(t, s):
            pltpu.sync_copy(idx_hbm.at[t, pl.ds(woff, bw)], idx_v[s])
            pltpu.async_copy(table_hbm.at[idx_v[s]], rows_v[s], gsem.at[s])

        def wait_gather(s):
            pltpu.make_async_copy(table_hbm.at[idx_v[s]], rows_v[s],
                                  gsem.at[s]).wait()

        def transpose(s):
            for cc in range(cb_per_w):
                ccv = jnp.full((_LANES,), cc, jnp.int32)

                def b_body(b8, carry, cc=cc, ccv=ccv):
                    for k in range(8):
                        b2 = b8 * 8 + k
                        row = 128 * cc + b2
                        dcv = jnp.full((_LANES,), b2, jnp.int32)
                        v0 = rows_v[s][row, pl.ds(0, _LANES)]
                        v1 = rows_v[s][row, pl.ds(_LANES, _LANES)]
                        plsc.store_scatter(panel_v[s], [ccv, iota_lo, dcv],
                                           v0)
                        plsc.store_scatter(panel_v[s], [ccv, iota_hi, dcv],
                                           v1)
                    return carry

                lax.fori_loop(0, 16, b_body, 0)

        def fire_out(t, s):
            for cc in range(cb_per_w):
                for r in range(n_dt):
                    pltpu.async_copy(
                        panel_v[s].at[cc, pl.ds(8 * r, 8), pl.ds(0, 128)],
                        out_hbm.at[t, r, cbase + cc], osem.at[s])

        def wait_out(t, s):
            for cc in range(cb_per_w):
                for r in range(n_dt):
                    pltpu.make_async_copy(
                        panel_v[s].at[cc, pl.ds(8 * r, 8), pl.ds(0, 128)],
                        out_hbm.at[t, r, cbase + cc], osem.at[s]).wait()

        for s in range(_NBUF):
            start(s, s)

        def body(g, carry):
            for s in range(_NBUF):
                t_proc = (g - 1) * _NBUF + s
                t_next = g * _NBUF + s
                wait_gather(s)

                @pl.when(g >= 2)
                def _():
                    wait_out(t_proc - _NBUF, s)

                transpose(s)
                fire_out(t_proc, s)
                start(t_next, s)
            return carry

        lax.fori_loop(1, n_groups, body, 0)

        for s in range(_NBUF):
            t_proc = (n_groups - 1) * _NBUF + s
            wait_gather(s)
            wait_out(t_proc - _NBUF, s)
            transpose(s)
            fire_out(t_proc, s)
        for s in range(_NBUF):
            t_proc = (n_groups - 1) * _NBUF + s
            wait_out(t_proc, s)

    return emb_kernel(table, ids_t)


def kernel(words_as_ids, embedding_table):
    batch, hist = words_as_ids.shape
    depth = embedding_table.shape[1]
    ids_t = words_as_ids.astype(jnp.int32).T
    num_v = embedding_table.shape[0]
    table_rm = _table_rows(embedding_table.T, 8192).reshape(num_v, depth)
    out5 = _embedding_lookup(table_rm, ids_t)
    out = out5.transpose(0, 1, 3, 2, 4).reshape(hist, depth, batch)
    return out.transpose(2, 0, 1)


# speedup vs baseline: 2.9362x; 1.0106x over previous
import functools

import jax
import jax.numpy as jnp
from jax import lax
from jax.experimental import pallas as pl
from jax.experimental.pallas import tpu as pltpu
from jax.experimental.pallas import tpu_sc as plsc

_NUM_CORES = 2
_NUM_SUBCORES = 16
_NUM_WORKERS = _NUM_CORES * _NUM_SUBCORES

_LANES = 16
_NBUF = 2


def _table_rows(table_t, block_v):
    depth, num_v = table_t.shape
    pack = 128 // depth
    grid = (num_v + block_v - 1) // block_v

    def body(x_ref, o_ref):
        t4 = x_ref[...].T.reshape(block_v // pack, pack, depth)
        for l in range(pack):
            o_ref[:, depth * l:depth * (l + 1)] = t4[:, l, :]

    return pl.pallas_call(
        body,
        grid=(grid,),
        in_specs=[pl.BlockSpec((depth, block_v), lambda j: (0, j))],
        out_specs=pl.BlockSpec((block_v // pack, 128), lambda j: (j, 0)),
        out_shape=jax.ShapeDtypeStruct((num_v * depth // 128, 128),
                                       table_t.dtype),
    )(table_t)


@jax.jit
def _embedding_lookup(table, ids_t):
    hist, batch = ids_t.shape
    depth = table.shape[1]
    n_dt = depth // 8
    n_cb = batch // 128
    cb_per_w = n_cb // _NUM_WORKERS
    bw = 128 * cb_per_w
    n_groups = hist // _NBUF

    mesh = plsc.VectorSubcoreMesh(
        core_axis_name="c",
        subcore_axis_name="s",
        num_cores=_NUM_CORES,
        num_subcores=_NUM_SUBCORES,
    )

    @functools.partial(
        pl.kernel,
        mesh=mesh,
        out_type=jax.ShapeDtypeStruct((hist, n_dt, n_cb, 8, 128),
                                      table.dtype),
        scratch_types=(
            [pltpu.VMEM((bw,), jnp.int32) for _ in range(_NBUF)]
            + [pltpu.VMEM((bw, depth), table.dtype) for _ in range(_NBUF)]
            + [pltpu.VMEM((cb_per_w, depth, 129), table.dtype)
               for _ in range(_NBUF)]
            + [pltpu.SemaphoreType.DMA((_NBUF,)),
               pltpu.SemaphoreType.DMA((_NBUF,))]
        ),
        compiler_params=pltpu.CompilerParams(use_tc_tiling_on_sc=False,
                                             needs_layout_passes=False),
    )
    def emb_kernel(table_hbm, idx_hbm, out_hbm, *scratch):
        idx_v = scratch[:_NBUF]
        rows_v = scratch[_NBUF:2 * _NBUF]
        panel_v = scratch[2 * _NBUF:3 * _NBUF]
        gsem, osem = scratch[3 * _NBUF], scratch[3 * _NBUF + 1]
        wid = lax.axis_index("s") * _NUM_CORES + lax.axis_index("c")
        woff = wid * bw
        cbase = wid * cb_per_w
        iota_lo = lax.iota(jnp.int32, _LANES)
        iota_hi = iota_lo + _LANES

        def start(t, s):
            pltpu.sync_copy(idx_hbm.at[t, pl.ds(woff, bw)], idx_v[s])
            pltpu.async_copy(table_hbm.at[idx_v[s]], rows_v[s], gsem.at[s])

        def wait_gather(s):
            pltpu.make_async_copy(table_hbm.at[idx_v[s]], rows_v[s],
                                  gsem.at[s]).wait()

        def transpose(s):
            for cc in range(cb_per_w):
                ccv = jnp.full((_LANES,), cc, jnp.int32)

                def b_body(b8, carry, cc=cc, ccv=ccv):
                    for k in range(8):
                        b2 = b8 * 8 + k
                        row = 128 * cc + b2
                        dcv = jnp.full((_LANES,), b2, jnp.int32)
                        v0 = rows_v[s][row, pl.ds(0, _LANES)]
                        v1 = rows_v[s][row, pl.ds(_LANES, _LANES)]
                        plsc.store_scatter(panel_v[s], [ccv, iota_lo, dcv],
                                           v0)
                        plsc.store_scatter(panel_v[s], [ccv, iota_hi, dcv],
                                           v1)
                    return carry

                lax.fori_loop(0, 16, b_body, 0)

        def fire_out(t, s):
            for cc in range(cb_per_w):
                for r in range(n_dt):
                    pltpu.async_copy(
                        panel_v[s].at[cc, pl.ds(8 * r, 8), pl.ds(0, 128)],
                        out_hbm.at[t, r, cbase + cc], osem.at[s])

        def wait_out(t, s):
            for cc in range(cb_per_w):
                for r in range(n_dt):
                    pltpu.make_async_copy(
                        panel_v[s].at[cc, pl.ds(8 * r, 8), pl.ds(0, 128)],
                        out_hbm.at[t, r, cbase + cc], osem.at[s]).wait()

        for s in range(_NBUF):
            start(s, s)

        def body(g, carry):
            for s in range(_NBUF):
                t_proc = (g - 1) * _NBUF + s
                t_next = g * _NBUF + s
                wait_gather(s)

                @pl.when(g >= 2)
                def _():
                    wait_out(t_proc - _NBUF, s)

                transpose(s)
                fire_out(t_proc, s)
                start(t_next, s)
            return carry

        lax.fori_loop(1, n_groups, body, 0)

        for s in range(_NBUF):
            t_proc = (n_groups - 1) * _NBUF + s
            wait_gather(s)
            wait_out(t_proc - _NBUF, s)
            transpose(s)
            fire_out(t_proc, s)
        for s in range(_NBUF):
            t_proc = (n_groups - 1) * _NBUF + s
            wait_out(t_proc, s)

    return emb_kernel(table, ids_t)


def kernel(words_as_ids, embedding_table):
    batch, hist = words_as_ids.shape
    depth = embedding_table.shape[1]
    ids_t = words_as_ids.astype(jnp.int32).T
    num_v = embedding_table.shape[0]
    table_rm = _table_rows(embedding_table.T, 16384).reshape(num_v, depth)
    out5 = _embedding_lookup(table_rm, ids_t)
    out = out5.transpose(0, 1, 3, 2, 4).reshape(hist, depth, batch)
    return out.transpose(2, 0, 1)
